# Initial kernel scaffold; baseline (speedup 1.0000x reference)
import jax
import jax.numpy as jnp
from jax.experimental import pallas as pl


def kernel(hidden_states, W):
    raise NotImplementedError("write your pallas kernel here")


# speedup vs baseline: 1.5056x; 1.5056x over previous
import jax
import jax.numpy as jnp
from jax.experimental import pallas as pl


def _router_kernel(x_ref, w_ref, s_ref, i_ref):
    x = x_ref[...]
    w = w_ref[...]
    logits = jax.lax.dot_general(
        x, w, (((1,), (1,)), ((), ())),
        preferred_element_type=jnp.float32)
    e = logits.shape[-1]
    m = jnp.max(logits, axis=-1, keepdims=True)
    z = jnp.sum(jnp.exp(logits - m), axis=-1, keepdims=True)
    iota = jax.lax.broadcasted_iota(jnp.int32, logits.shape, 1)
    big = jnp.int32(e)
    idx1 = jnp.min(jnp.where(logits == m, iota, big), axis=-1, keepdims=True)
    masked = jnp.where(iota == idx1, -jnp.inf, logits)
    m2 = jnp.max(masked, axis=-1, keepdims=True)
    idx2 = jnp.min(jnp.where(masked == m2, iota, big), axis=-1, keepdims=True)
    s1 = 1.0 / z
    s2 = jnp.exp(m2 - m) / z
    s_ref[...] = jnp.concatenate([s1, s2], axis=-1)
    i_ref[...] = jnp.concatenate([idx1, idx2], axis=-1)


def kernel(hidden_states, W):
    B, S, D = hidden_states.shape
    E = W.shape[0]
    N = B * S
    x = hidden_states.reshape(N, D)
    T = 512
    scores, indices = pl.pallas_call(
        _router_kernel,
        grid=(N // T,),
        in_specs=[
            pl.BlockSpec((T, D), lambda i: (i, 0)),
            pl.BlockSpec((E, D), lambda i: (0, 0)),
        ],
        out_specs=[
            pl.BlockSpec((T, 2), lambda i: (i, 0)),
            pl.BlockSpec((T, 2), lambda i: (i, 0)),
        ],
        out_shape=[
            jax.ShapeDtypeStruct((N, 2), jnp.float32),
            jax.ShapeDtypeStruct((N, 2), jnp.int32),
        ],
    )(x, W)
    return scores.reshape(B, S, 2), indices.reshape(B, S, 2)


# speedup vs baseline: 1.7662x; 1.1731x over previous
import jax
import jax.numpy as jnp
from jax.experimental import pallas as pl
from jax.experimental.pallas import tpu as pltpu


def _router_kernel(x_ref, w_ref, s_ref, i_ref):
    x = x_ref[...]
    w = w_ref[...]
    logits = jax.lax.dot_general(
        x, w, (((1,), (1,)), ((), ())),
        preferred_element_type=jnp.float32)
    e = logits.shape[-1]
    m = jnp.max(logits, axis=-1, keepdims=True)
    z = jnp.sum(jnp.exp(logits - m), axis=-1, keepdims=True)
    iota = jax.lax.broadcasted_iota(jnp.int32, logits.shape, 1)
    big = jnp.int32(e)
    idx1 = jnp.min(jnp.where(logits == m, iota, big), axis=-1, keepdims=True)
    masked = jnp.where(iota == idx1, -jnp.inf, logits)
    m2 = jnp.max(masked, axis=-1, keepdims=True)
    idx2 = jnp.min(jnp.where(masked == m2, iota, big), axis=-1, keepdims=True)
    s1 = 1.0 / z
    s2 = jnp.exp(m2 - m) / z
    s_ref[...] = jnp.concatenate([s1, s2], axis=-1)
    i_ref[...] = jnp.concatenate([idx1, idx2], axis=-1)


def kernel(hidden_states, W):
    B, S, D = hidden_states.shape
    E = W.shape[0]
    N = B * S
    x = hidden_states.reshape(N, D)
    T = 1024
    scores, indices = pl.pallas_call(
        _router_kernel,
        grid=(N // T,),
        compiler_params=pltpu.CompilerParams(
            dimension_semantics=("parallel",)),
        in_specs=[
            pl.BlockSpec((T, D), lambda i: (i, 0)),
            pl.BlockSpec((E, D), lambda i: (0, 0)),
        ],
        out_specs=[
            pl.BlockSpec((T, 2), lambda i: (i, 0)),
            pl.BlockSpec((T, 2), lambda i: (i, 0)),
        ],
        out_shape=[
            jax.ShapeDtypeStruct((N, 2), jnp.float32),
            jax.ShapeDtypeStruct((N, 2), jnp.int32),
        ],
    )(x, W)
    return scores.reshape(B, S, 2), indices.reshape(B, S, 2)


# speedup vs baseline: 1.8441x; 1.0441x over previous
import jax
import jax.numpy as jnp
from jax.experimental import pallas as pl
from jax.experimental.pallas import tpu as pltpu


def _router_kernel(x_ref, w_ref, s_ref, i_ref):
    x = x_ref[...]
    w = w_ref[...]
    logits = jax.lax.dot_general(
        x, w, (((1,), (1,)), ((), ())),
        preferred_element_type=jnp.float32)
    e = logits.shape[-1]
    m = jnp.max(logits, axis=-1, keepdims=True)
    z = jnp.sum(jnp.exp(logits - m), axis=-1, keepdims=True)
    iota = jax.lax.broadcasted_iota(jnp.int32, logits.shape, 1)
    big = jnp.int32(e)
    idx1 = jnp.min(jnp.where(logits == m, iota, big), axis=-1, keepdims=True)
    masked = jnp.where(iota == idx1, -jnp.inf, logits)
    m2 = jnp.max(masked, axis=-1, keepdims=True)
    idx2 = jnp.min(jnp.where(masked == m2, iota, big), axis=-1, keepdims=True)
    s1 = 1.0 / z
    s2 = jnp.exp(m2 - m) / z
    s_ref[...] = jnp.concatenate([s1, s2], axis=-1)
    i_ref[...] = jnp.concatenate([idx1, idx2], axis=-1)


def kernel(hidden_states, W):
    B, S, D = hidden_states.shape
    E = W.shape[0]
    N = B * S
    x = hidden_states.reshape(N, D)
    T = 2048
    scores, indices = pl.pallas_call(
        _router_kernel,
        grid=(N // T,),
        compiler_params=pltpu.CompilerParams(
            dimension_semantics=("parallel",)),
        in_specs=[
            pl.BlockSpec((T, D), lambda i: (i, 0)),
            pl.BlockSpec((E, D), lambda i: (0, 0)),
        ],
        out_specs=[
            pl.BlockSpec((T, 2), lambda i: (i, 0)),
            pl.BlockSpec((T, 2), lambda i: (i, 0)),
        ],
        out_shape=[
            jax.ShapeDtypeStruct((N, 2), jnp.float32),
            jax.ShapeDtypeStruct((N, 2), jnp.int32),
        ],
    )(x, W)
    return scores.reshape(B, S, 2), indices.reshape(B, S, 2)
